# baseline (speedup 1.0000x reference)
import jax
import jax.numpy as jnp
from jax import lax
from jax.experimental import pallas as pl
from jax.experimental.pallas import tpu as pltpu
from jax.experimental.pallas import tpu_sc as plsc

_B, _D = 8192, 2048
_N_IN = 4
_NC, _NS = 2, 16
_NW = _NC * _NS
_ROWS_W = _B // _NW
_NDMA = 2
_ROWS_DMA = _ROWS_W // _NDMA


def _mux_body(x0, x1, x2, x3, sel_hbm, out, sel_v, sem):
    xs = (x0, x1, x2, x3)

    wid = lax.axis_index("s") * _NC + lax.axis_index("c")
    base = wid * _ROWS_W

    pltpu.sync_copy(sel_hbm, sel_v)
    selv = sel_v[...]
    preds = [jnp.any(selv == i) for i in range(_N_IN)]

    def rows(d):
        return pl.ds(base + d * _ROWS_DMA, _ROWS_DMA)

    for i in range(_N_IN):
        @pl.when(preds[i])
        def _(i=i):
            for d in range(_NDMA):
                pltpu.async_copy(xs[i].at[rows(d)], out.at[rows(d)], sem)
    for d in range(_NDMA):
        pltpu.make_async_copy(xs[0].at[rows(d)], out.at[rows(d)], sem).wait()


def kernel(x0, x1, x2, x3, sel):
    sel_arr = jnp.full((16,), sel, dtype=jnp.int32)
    mesh = plsc.VectorSubcoreMesh(
        core_axis_name="c", subcore_axis_name="s",
        num_cores=_NC, num_subcores=_NS)
    mux = pl.kernel(
        _mux_body,
        out_type=jax.ShapeDtypeStruct((_B, _D), jnp.float32),
        mesh=mesh,
        compiler_params=pltpu.CompilerParams(needs_layout_passes=False),
        scratch_types=(
            pltpu.VMEM((16,), jnp.int32),
            pltpu.SemaphoreType.DMA,
        ),
    )
    return mux(x0, x1, x2, x3, sel_arr)


# speedup vs baseline: 30.9513x; 30.9513x over previous
import jax
import jax.numpy as jnp
from jax import lax
from jax.experimental import pallas as pl
from jax.experimental.pallas import tpu as pltpu
from jax.experimental.pallas import tpu_sc as plsc

_B, _D = 8192, 2048
_N_IN = 4
_NC, _NS = 2, 16
_NW = _NC * _NS
_ROWS_W = _B // _NW
_CHUNK = 16
_NCH = _ROWS_W // _CHUNK
_NBUF = 3


def _mux_body(x0, x1, x2, x3, sel_hbm, out,
              sel_v, stage, r0, r1, r2, w0, w1, w2):
    xs = (x0, x1, x2, x3)
    rsems = (r0, r1, r2)
    wsems = (w0, w1, w2)

    sid = lax.axis_index("s")
    wid = sid * _NC + lax.axis_index("c")
    base = wid * _ROWS_W

    pltpu.sync_copy(sel_hbm, sel_v)
    selv = sel_v[...]
    preds = [jnp.any(selv == i) for i in range(_N_IN)]

    def rows(c):
        return pl.ds(base + c * _CHUNK, _CHUNK)

    def buf(k):
        return stage.at[sid, k]

    def start_read(c):
        k = c % _NBUF
        for i in range(_N_IN):
            @pl.when(preds[i])
            def _(i=i, k=k, c=c):
                pltpu.async_copy(xs[i].at[rows(c)], buf(k), rsems[k])

    def wait_read(c):
        k = c % _NBUF
        pltpu.make_async_copy(xs[0].at[rows(c)], buf(k), rsems[k]).wait()

    def start_write(c):
        k = c % _NBUF
        pltpu.async_copy(buf(k), out.at[rows(c)], wsems[k])

    def wait_write(c):
        k = c % _NBUF
        pltpu.make_async_copy(buf(k), out.at[rows(c)], wsems[k]).wait()

    for c in range(_NBUF):
        start_read(c)

    pending_writes = []
    for c in range(_NCH):
        wait_read(c)
        start_write(c)
        pending_writes.append(c)
        nxt = c + _NBUF
        if nxt < _NCH:
            wait_write(pending_writes.pop(0))
            start_read(nxt)
    for c in pending_writes:
        wait_write(c)


def kernel(x0, x1, x2, x3, sel):
    sel_arr = jnp.full((16,), sel, dtype=jnp.int32)
    mesh = plsc.VectorSubcoreMesh(
        core_axis_name="c", subcore_axis_name="s",
        num_cores=_NC, num_subcores=_NS)
    mux = pl.kernel(
        _mux_body,
        out_type=jax.ShapeDtypeStruct((_B, _D), jnp.float32),
        mesh=mesh,
        compiler_params=pltpu.CompilerParams(needs_layout_passes=False),
        scratch_types=(
            [pltpu.VMEM((16,), jnp.int32),
             pltpu.MemorySpace.VMEM_SHARED((_NS, _NBUF, _CHUNK, _D),
                                           jnp.float32)]
            + [pltpu.SemaphoreType.DMA for _ in range(2 * _NBUF)]
        ),
    )
    return mux(x0, x1, x2, x3, sel_arr)
